# baseline (speedup 1.0000x reference)
import functools

import jax
import jax.numpy as jnp
from jax import lax
from jax.experimental import pallas as pl
from jax.experimental.pallas import tpu as pltpu
from jax.experimental.pallas import tpu_sc as plsc

_NUM_BINS = 1024
_NSUB = 16
_BPW = _NUM_BINS // _NSUB
_LANES = 128
_BLK = 2048


def _tc_body(beta_ref, p_ref, t_ref, pid_ref, rec_ref, w_ref, wm_ref, c_ref):
    beta = beta_ref[...]
    d = p_ref[...] - t_ref[...]
    mse = jnp.sum(d * d, axis=1)
    m = (pid_ref[...] > 0) & (rec_ref[...] > 0)
    ath = 0.5 * jnp.log((1.0 + beta) / (1.0 - beta))
    w = jnp.where(m, ath * ath, 0.0)
    w_ref[...] = w
    wm_ref[...] = w * mse
    c_ref[...] = m.astype(jnp.float32)


def _sc_body(rows, w_hbm, wm_hbm, c_hbm, pid_hbm,
             aw_hbm, am_hbm, ac_hbm,
             idx_v, wv, mv, cv, zb, ow, om, oc,
             aw, am, ac, sem):
    s = lax.axis_index("s")
    zero16 = jnp.zeros((16,), jnp.float32)

    for k in range(_BPW // 16):
        zb[pl.ds(16 * k, 16)] = zero16
    pltpu.sync_copy(zb, aw.at[pl.ds(s * _BPW, _BPW)])
    pltpu.sync_copy(zb, am.at[pl.ds(s * _BPW, _BPW)])
    pltpu.sync_copy(zb, ac.at[pl.ds(s * _BPW, _BPW)])

    sl = pl.ds(s * rows, rows)
    cp1 = pltpu.async_copy(pid_hbm.at[sl], idx_v, sem)
    cp2 = pltpu.async_copy(w_hbm.at[sl], wv, sem)
    cp3 = pltpu.async_copy(wm_hbm.at[sl], mv, sem)
    cp4 = pltpu.async_copy(c_hbm.at[sl], cv, sem)
    cp1.wait()
    cp2.wait()
    cp3.wait()
    cp4.wait()
    plsc.subcore_barrier()

    for j in range(rows):
        ji = idx_v.at[j]
        pltpu.sync_copy(wv.at[j], aw.at[ji], add=True)
        pltpu.sync_copy(mv.at[j], am.at[ji], add=True)
        pltpu.sync_copy(cv.at[j], ac.at[ji], add=True)
    plsc.subcore_barrier()

    pltpu.sync_copy(aw.at[pl.ds(s * _BPW, _BPW)], ow)
    pltpu.sync_copy(am.at[pl.ds(s * _BPW, _BPW)], om)
    pltpu.sync_copy(ac.at[pl.ds(s * _BPW, _BPW)], oc)
    pltpu.sync_copy(ow, aw_hbm.at[pl.ds(s * _BPW, _BPW)])
    pltpu.sync_copy(om, am_hbm.at[pl.ds(s * _BPW, _BPW)])
    pltpu.sync_copy(oc, ac_hbm.at[pl.ds(s * _BPW, _BPW)])


def _fin_body(aw_ref, am_ref, ac_ref, o_ref):
    aw = aw_ref[...]
    am = am_ref[...]
    ac = ac_ref[...]
    pres = ac > 0.0
    safe = jnp.where(pres, aw, 1.0)
    ratios = jnp.where(pres, am / safe, 0.0)
    count = jnp.sum(pres.astype(jnp.float32))
    o_ref[...] = jnp.full((1, 1), 100.0 * jnp.sum(ratios) / count)


@jax.jit
def kernel(beta, pred, particle_id, track_params, reconstructable):
    n = beta.shape[0]
    d = pred.shape[1]
    grain = _NSUB * _LANES
    rows = ((n + grain - 1) // grain + 7) // 8 * 8
    npad = rows * grain
    padn = npad - n

    beta_p = jnp.pad(beta, (0, padn))
    pid_p = jnp.pad(particle_id.astype(jnp.int32), (0, padn))
    rec_p = jnp.pad(reconstructable.astype(jnp.int32), (0, padn))
    pred_p = jnp.pad(pred, ((0, padn), (0, 0)))
    track_p = jnp.pad(track_params, ((0, padn), (0, 0)))

    w, wm, c = pl.pallas_call(
        _tc_body,
        grid=(npad // _BLK,),
        in_specs=[
            pl.BlockSpec((_BLK,), lambda i: (i,)),
            pl.BlockSpec((_BLK, d), lambda i: (i, 0)),
            pl.BlockSpec((_BLK, d), lambda i: (i, 0)),
            pl.BlockSpec((_BLK,), lambda i: (i,)),
            pl.BlockSpec((_BLK,), lambda i: (i,)),
        ],
        out_specs=[pl.BlockSpec((_BLK,), lambda i: (i,))] * 3,
        out_shape=[jax.ShapeDtypeStruct((npad,), jnp.float32)] * 3,
    )(beta_p, pred_p, track_p, pid_p, rec_p)

    nrow = npad // _LANES
    mesh = plsc.VectorSubcoreMesh(
        core_axis_name="c", subcore_axis_name="s", num_cores=1
    )
    sc = pl.kernel(
        functools.partial(_sc_body, rows),
        out_type=[jax.ShapeDtypeStruct((_NUM_BINS,), jnp.float32)] * 3,
        mesh=mesh,
        scratch_types=[
            pltpu.VMEM((rows, _LANES), jnp.int32),
            pltpu.VMEM((rows, _LANES), jnp.float32),
            pltpu.VMEM((rows, _LANES), jnp.float32),
            pltpu.VMEM((rows, _LANES), jnp.float32),
            pltpu.VMEM((_BPW,), jnp.float32),
            pltpu.VMEM((_BPW,), jnp.float32),
            pltpu.VMEM((_BPW,), jnp.float32),
            pltpu.VMEM((_BPW,), jnp.float32),
            pltpu.VMEM_SHARED((_NUM_BINS,), jnp.float32),
            pltpu.VMEM_SHARED((_NUM_BINS,), jnp.float32),
            pltpu.VMEM_SHARED((_NUM_BINS,), jnp.float32),
            pltpu.SemaphoreType.DMA,
        ],
    )
    aw, am, ac = sc(
        w.reshape(nrow, _LANES),
        wm.reshape(nrow, _LANES),
        c.reshape(nrow, _LANES),
        pid_p.reshape(nrow, _LANES),
    )

    out = pl.pallas_call(
        _fin_body,
        out_shape=jax.ShapeDtypeStruct((1, 1), jnp.float32),
    )(aw, am, ac)
    return out[0, 0]


# speedup vs baseline: 2.2438x; 2.2438x over previous
import functools

import jax
import jax.numpy as jnp
from jax import lax
from jax.experimental import pallas as pl
from jax.experimental.pallas import tpu as pltpu
from jax.experimental.pallas import tpu_sc as plsc

_NUM_BINS = 1024
_NSUB = 16
_BPW = _NUM_BINS // _NSUB
_LANES = 128
_BLK = 2048


def _tc_body(beta_ref, p_ref, t_ref, pid_ref, rec_ref, w_ref, wm_ref, c_ref):
    beta = beta_ref[...]
    d = p_ref[...] - t_ref[...]
    mse = jnp.sum(d * d, axis=0)
    m = (pid_ref[...] > 0) & (rec_ref[...] > 0)
    ath = 0.5 * jnp.log((1.0 + beta) / (1.0 - beta))
    w = jnp.where(m, ath * ath, 0.0)
    w_ref[...] = w
    wm_ref[...] = w * mse
    c_ref[...] = m.astype(jnp.float32)


def _sc_body(rows, w_hbm, wm_hbm, c_hbm, pid_hbm,
             aw_hbm, am_hbm, ac_hbm,
             idx_v, wv, mv, cv, zb, ow, om, oc,
             aw, am, ac, sem):
    s = lax.axis_index("s")
    zero16 = jnp.zeros((16,), jnp.float32)

    for k in range(_BPW // 16):
        zb[pl.ds(16 * k, 16)] = zero16
    pltpu.sync_copy(zb, aw.at[pl.ds(s * _BPW, _BPW)])
    pltpu.sync_copy(zb, am.at[pl.ds(s * _BPW, _BPW)])
    pltpu.sync_copy(zb, ac.at[pl.ds(s * _BPW, _BPW)])

    sl = pl.ds(s * rows, rows)
    cp1 = pltpu.async_copy(pid_hbm.at[sl], idx_v, sem)
    cp2 = pltpu.async_copy(w_hbm.at[sl], wv, sem)
    cp3 = pltpu.async_copy(wm_hbm.at[sl], mv, sem)
    cp4 = pltpu.async_copy(c_hbm.at[sl], cv, sem)
    cp1.wait()
    cp2.wait()
    cp3.wait()
    cp4.wait()
    plsc.subcore_barrier()

    for j in range(rows):
        ji = idx_v.at[j]
        pltpu.sync_copy(wv.at[j], aw.at[ji], add=True)
        pltpu.sync_copy(mv.at[j], am.at[ji], add=True)
        pltpu.sync_copy(cv.at[j], ac.at[ji], add=True)
    plsc.subcore_barrier()

    pltpu.sync_copy(aw.at[pl.ds(s * _BPW, _BPW)], ow)
    pltpu.sync_copy(am.at[pl.ds(s * _BPW, _BPW)], om)
    pltpu.sync_copy(ac.at[pl.ds(s * _BPW, _BPW)], oc)
    pltpu.sync_copy(ow, aw_hbm.at[pl.ds(s * _BPW, _BPW)])
    pltpu.sync_copy(om, am_hbm.at[pl.ds(s * _BPW, _BPW)])
    pltpu.sync_copy(oc, ac_hbm.at[pl.ds(s * _BPW, _BPW)])


def _fin_body(aw_ref, am_ref, ac_ref, o_ref):
    aw = aw_ref[...]
    am = am_ref[...]
    ac = ac_ref[...]
    pres = ac > 0.0
    safe = jnp.where(pres, aw, 1.0)
    ratios = jnp.where(pres, am / safe, 0.0)
    count = jnp.sum(pres.astype(jnp.float32))
    o_ref[...] = jnp.full((1, 1), 100.0 * jnp.sum(ratios) / count)


@jax.jit
def kernel(beta, pred, particle_id, track_params, reconstructable):
    n = beta.shape[0]
    d = pred.shape[1]
    grain = _NSUB * _LANES
    rows = ((n + grain - 1) // grain + 7) // 8 * 8
    npad = rows * grain
    padn = npad - n

    beta_p = jnp.pad(beta, (0, padn))
    pid_p = jnp.pad(particle_id.astype(jnp.int32), (0, padn))
    rec_p = jnp.pad(reconstructable.astype(jnp.int32), (0, padn))
    pred_t = jnp.pad(pred, ((0, padn), (0, 0))).T
    track_t = jnp.pad(track_params, ((0, padn), (0, 0))).T

    w, wm, c = pl.pallas_call(
        _tc_body,
        out_shape=[jax.ShapeDtypeStruct((npad,), jnp.float32)] * 3,
    )(beta_p, pred_t, track_t, pid_p, rec_p)

    nrow = npad // _LANES
    mesh = plsc.VectorSubcoreMesh(
        core_axis_name="c", subcore_axis_name="s", num_cores=1
    )
    sc = pl.kernel(
        functools.partial(_sc_body, rows),
        out_type=[jax.ShapeDtypeStruct((_NUM_BINS,), jnp.float32)] * 3,
        mesh=mesh,
        scratch_types=[
            pltpu.VMEM((rows, _LANES), jnp.int32),
            pltpu.VMEM((rows, _LANES), jnp.float32),
            pltpu.VMEM((rows, _LANES), jnp.float32),
            pltpu.VMEM((rows, _LANES), jnp.float32),
            pltpu.VMEM((_BPW,), jnp.float32),
            pltpu.VMEM((_BPW,), jnp.float32),
            pltpu.VMEM((_BPW,), jnp.float32),
            pltpu.VMEM((_BPW,), jnp.float32),
            pltpu.VMEM_SHARED((_NUM_BINS,), jnp.float32),
            pltpu.VMEM_SHARED((_NUM_BINS,), jnp.float32),
            pltpu.VMEM_SHARED((_NUM_BINS,), jnp.float32),
            pltpu.SemaphoreType.DMA,
        ],
    )
    aw, am, ac = sc(
        w.reshape(nrow, _LANES),
        wm.reshape(nrow, _LANES),
        c.reshape(nrow, _LANES),
        pid_p.reshape(nrow, _LANES),
    )

    out = pl.pallas_call(
        _fin_body,
        out_shape=jax.ShapeDtypeStruct((1, 1), jnp.float32),
    )(aw, am, ac)
    return out[0, 0]


# speedup vs baseline: 4.5318x; 2.0197x over previous
import functools

import jax
import jax.numpy as jnp
from jax import lax
from jax.experimental import pallas as pl
from jax.experimental.pallas import tpu as pltpu
from jax.experimental.pallas import tpu_sc as plsc

_NUM_BINS = 1024
_LANES = 128

_INFO = plsc.get_sparse_core_info()
_NC = _INFO.num_cores
_NS = _INFO.num_subcores
_NW = _NC * _NS
_BPW = _NUM_BINS // _NS


def _tc_body(beta_ref, p_ref, t_ref, pid_ref, rec_ref, w_ref, wm_ref, c_ref):
    beta = beta_ref[...]
    d = p_ref[...] - t_ref[...]
    mse = jnp.sum(d * d, axis=0)
    m = (pid_ref[...] > 0) & (rec_ref[...] > 0)
    ath = 0.5 * jnp.log((1.0 + beta) / (1.0 - beta))
    w = jnp.where(m, ath * ath, 0.0)
    w_ref[...] = w
    wm_ref[...] = w * mse
    c_ref[...] = m.astype(jnp.float32)


def _sc_body(rows, w_hbm, wm_hbm, c_hbm, pid_hbm,
             aw_hbm, am_hbm, ac_hbm,
             idx_v, wv, mv, cv, zb, ow, om, oc,
             aw, am, ac):
    c = lax.axis_index("c")
    s = lax.axis_index("s")
    wid = s * _NC + c
    chunk = rows * _LANES
    zero16 = jnp.zeros((16,), jnp.float32)

    for k in range(_BPW // 16):
        zb[pl.ds(16 * k, 16)] = zero16
    pltpu.sync_copy(zb, aw.at[pl.ds(s * _BPW, _BPW)])
    pltpu.sync_copy(zb, am.at[pl.ds(s * _BPW, _BPW)])
    pltpu.sync_copy(zb, ac.at[pl.ds(s * _BPW, _BPW)])

    pltpu.sync_copy(pid_hbm.at[pl.ds(wid * chunk, chunk)], idx_v)
    pltpu.sync_copy(w_hbm.at[pl.ds(wid * chunk, chunk)], wv)
    pltpu.sync_copy(wm_hbm.at[pl.ds(wid * chunk, chunk)], mv)
    pltpu.sync_copy(c_hbm.at[pl.ds(wid * chunk, chunk)], cv)
    plsc.subcore_barrier()

    for j in range(rows):
        ji = idx_v.at[pl.ds(j * _LANES, _LANES)]
        pltpu.sync_copy(wv.at[pl.ds(j * _LANES, _LANES)], aw.at[ji], add=True)
        pltpu.sync_copy(mv.at[pl.ds(j * _LANES, _LANES)], am.at[ji], add=True)
        pltpu.sync_copy(cv.at[pl.ds(j * _LANES, _LANES)], ac.at[ji], add=True)
    plsc.subcore_barrier()

    off = c * _NUM_BINS + s * _BPW
    pltpu.sync_copy(aw.at[pl.ds(s * _BPW, _BPW)], ow)
    pltpu.sync_copy(am.at[pl.ds(s * _BPW, _BPW)], om)
    pltpu.sync_copy(ac.at[pl.ds(s * _BPW, _BPW)], oc)
    pltpu.sync_copy(ow, aw_hbm.at[pl.ds(off, _BPW)])
    pltpu.sync_copy(om, am_hbm.at[pl.ds(off, _BPW)])
    pltpu.sync_copy(oc, ac_hbm.at[pl.ds(off, _BPW)])


def _fin_body(aw_ref, am_ref, ac_ref, o_ref):
    aw2 = aw_ref[...]
    am2 = am_ref[...]
    ac2 = ac_ref[...]
    aw = aw2[:_NUM_BINS]
    am = am2[:_NUM_BINS]
    ac = ac2[:_NUM_BINS]
    for k in range(1, _NC):
        aw = aw + aw2[k * _NUM_BINS:(k + 1) * _NUM_BINS]
        am = am + am2[k * _NUM_BINS:(k + 1) * _NUM_BINS]
        ac = ac + ac2[k * _NUM_BINS:(k + 1) * _NUM_BINS]
    pres = ac > 0.0
    safe = jnp.where(pres, aw, 1.0)
    ratios = jnp.where(pres, am / safe, 0.0)
    count = jnp.sum(pres.astype(jnp.float32))
    o_ref[...] = jnp.full((1, 1), 100.0 * jnp.sum(ratios) / count)


@jax.jit
def kernel(beta, pred, particle_id, track_params, reconstructable):
    n = beta.shape[0]
    grain = _NW * _LANES
    npad = ((n + grain - 1) // grain) * grain
    rows = npad // grain
    chunk = npad // _NW
    padn = npad - n

    beta_p = jnp.pad(beta, (0, padn))
    pid_p = jnp.pad(particle_id.astype(jnp.int32), (0, padn))
    rec_p = jnp.pad(reconstructable.astype(jnp.int32), (0, padn))
    pred_t = jnp.pad(pred, ((0, padn), (0, 0))).T
    track_t = jnp.pad(track_params, ((0, padn), (0, 0))).T

    w, wm, c = pl.pallas_call(
        _tc_body,
        out_shape=[jax.ShapeDtypeStruct((npad,), jnp.float32)] * 3,
    )(beta_p, pred_t, track_t, pid_p, rec_p)

    mesh = plsc.VectorSubcoreMesh(core_axis_name="c", subcore_axis_name="s")
    sc = pl.kernel(
        functools.partial(_sc_body, rows),
        out_type=[jax.ShapeDtypeStruct((_NC * _NUM_BINS,), jnp.float32)] * 3,
        mesh=mesh,
        scratch_types=[
            pltpu.VMEM((chunk,), jnp.int32),
            pltpu.VMEM((chunk,), jnp.float32),
            pltpu.VMEM((chunk,), jnp.float32),
            pltpu.VMEM((chunk,), jnp.float32),
            pltpu.VMEM((_BPW,), jnp.float32),
            pltpu.VMEM((_BPW,), jnp.float32),
            pltpu.VMEM((_BPW,), jnp.float32),
            pltpu.VMEM((_BPW,), jnp.float32),
            pltpu.VMEM_SHARED((_NUM_BINS,), jnp.float32),
            pltpu.VMEM_SHARED((_NUM_BINS,), jnp.float32),
            pltpu.VMEM_SHARED((_NUM_BINS,), jnp.float32),
        ],
    )
    aw, am, ac = sc(w, wm, c, pid_p)

    out = pl.pallas_call(
        _fin_body,
        out_shape=jax.ShapeDtypeStruct((1, 1), jnp.float32),
    )(aw, am, ac)
    return out[0, 0]


# speedup vs baseline: 5.8196x; 1.2842x over previous
import functools

import jax
import jax.numpy as jnp
from jax import lax
from jax.experimental import pallas as pl
from jax.experimental.pallas import tpu as pltpu
from jax.experimental.pallas import tpu_sc as plsc

_NUM_BINS = 1024
_LANES = 128

_INFO = plsc.get_sparse_core_info()
_NC = _INFO.num_cores
_NS = _INFO.num_subcores
_NW = _NC * _NS
_BPW = _NUM_BINS // _NS


def _tc_body(beta_ref, p_ref, t_ref, pid_ref, rec_ref,
             w_ref, wm_ref, c_ref, pid_out_ref):
    n = beta_ref.shape[0]
    npad = w_ref.shape[0]
    w_ref[...] = jnp.zeros((npad,), jnp.float32)
    wm_ref[...] = jnp.zeros((npad,), jnp.float32)
    c_ref[...] = jnp.zeros((npad,), jnp.float32)
    pid_out_ref[...] = jnp.zeros((npad,), jnp.int32)
    beta = beta_ref[...]
    d = p_ref[...] - t_ref[...]
    mse = jnp.sum(d * d, axis=0)
    pid = pid_ref[...]
    m = (pid > 0) & (rec_ref[...] > 0)
    ath = 0.5 * jnp.log((1.0 + beta) / (1.0 - beta))
    w = jnp.where(m, ath * ath, 0.0)
    w_ref[pl.ds(0, n)] = w
    wm_ref[pl.ds(0, n)] = w * mse
    c_ref[pl.ds(0, n)] = m.astype(jnp.float32)
    pid_out_ref[pl.ds(0, n)] = pid


def _sc_body(rows, w_hbm, wm_hbm, c_hbm, pid_hbm,
             aw_hbm, am_hbm, ac_hbm,
             idx_v, wv, mv, cv, zb, ow, om, oc,
             aw, am, ac):
    c = lax.axis_index("c")
    s = lax.axis_index("s")
    wid = s * _NC + c
    chunk = rows * _LANES
    zero16 = jnp.zeros((16,), jnp.float32)

    for k in range(_BPW // 16):
        zb[pl.ds(16 * k, 16)] = zero16
    pltpu.sync_copy(zb, aw.at[pl.ds(s * _BPW, _BPW)])
    pltpu.sync_copy(zb, am.at[pl.ds(s * _BPW, _BPW)])
    pltpu.sync_copy(zb, ac.at[pl.ds(s * _BPW, _BPW)])

    pltpu.sync_copy(pid_hbm.at[pl.ds(wid * chunk, chunk)], idx_v)
    pltpu.sync_copy(w_hbm.at[pl.ds(wid * chunk, chunk)], wv)
    pltpu.sync_copy(wm_hbm.at[pl.ds(wid * chunk, chunk)], mv)
    pltpu.sync_copy(c_hbm.at[pl.ds(wid * chunk, chunk)], cv)
    plsc.subcore_barrier()

    for j in range(rows):
        ji = idx_v.at[pl.ds(j * _LANES, _LANES)]
        pltpu.sync_copy(wv.at[pl.ds(j * _LANES, _LANES)], aw.at[ji], add=True)
        pltpu.sync_copy(mv.at[pl.ds(j * _LANES, _LANES)], am.at[ji], add=True)
        pltpu.sync_copy(cv.at[pl.ds(j * _LANES, _LANES)], ac.at[ji], add=True)
    plsc.subcore_barrier()

    off = c * _NUM_BINS + s * _BPW
    pltpu.sync_copy(aw.at[pl.ds(s * _BPW, _BPW)], ow)
    pltpu.sync_copy(am.at[pl.ds(s * _BPW, _BPW)], om)
    pltpu.sync_copy(ac.at[pl.ds(s * _BPW, _BPW)], oc)
    pltpu.sync_copy(ow, aw_hbm.at[pl.ds(off, _BPW)])
    pltpu.sync_copy(om, am_hbm.at[pl.ds(off, _BPW)])
    pltpu.sync_copy(oc, ac_hbm.at[pl.ds(off, _BPW)])


def _fin_body(aw_ref, am_ref, ac_ref, o_ref):
    aw2 = aw_ref[...]
    am2 = am_ref[...]
    ac2 = ac_ref[...]
    aw = aw2[:_NUM_BINS]
    am = am2[:_NUM_BINS]
    ac = ac2[:_NUM_BINS]
    for k in range(1, _NC):
        aw = aw + aw2[k * _NUM_BINS:(k + 1) * _NUM_BINS]
        am = am + am2[k * _NUM_BINS:(k + 1) * _NUM_BINS]
        ac = ac + ac2[k * _NUM_BINS:(k + 1) * _NUM_BINS]
    pres = ac > 0.0
    safe = jnp.where(pres, aw, 1.0)
    ratios = jnp.where(pres, am / safe, 0.0)
    count = jnp.sum(pres.astype(jnp.float32))
    o_ref[...] = jnp.full((1, 1), 100.0 * jnp.sum(ratios) / count)


@jax.jit
def kernel(beta, pred, particle_id, track_params, reconstructable):
    n = beta.shape[0]
    grain = _NW * _LANES
    npad = ((n + grain - 1) // grain) * grain
    rows = npad // grain
    chunk = npad // _NW
    padn = npad - n

    pid_i = particle_id.astype(jnp.int32)
    rec_i = reconstructable.astype(jnp.int32)

    w, wm, c, pid_p = pl.pallas_call(
        _tc_body,
        out_shape=[jax.ShapeDtypeStruct((npad,), jnp.float32)] * 3
        + [jax.ShapeDtypeStruct((npad,), jnp.int32)],
    )(beta, pred.T, track_params.T, pid_i, rec_i)

    mesh = plsc.VectorSubcoreMesh(core_axis_name="c", subcore_axis_name="s")
    sc = pl.kernel(
        functools.partial(_sc_body, rows),
        out_type=[jax.ShapeDtypeStruct((_NC * _NUM_BINS,), jnp.float32)] * 3,
        mesh=mesh,
        scratch_types=[
            pltpu.VMEM((chunk,), jnp.int32),
            pltpu.VMEM((chunk,), jnp.float32),
            pltpu.VMEM((chunk,), jnp.float32),
            pltpu.VMEM((chunk,), jnp.float32),
            pltpu.VMEM((_BPW,), jnp.float32),
            pltpu.VMEM((_BPW,), jnp.float32),
            pltpu.VMEM((_BPW,), jnp.float32),
            pltpu.VMEM((_BPW,), jnp.float32),
            pltpu.VMEM_SHARED((_NUM_BINS,), jnp.float32),
            pltpu.VMEM_SHARED((_NUM_BINS,), jnp.float32),
            pltpu.VMEM_SHARED((_NUM_BINS,), jnp.float32),
        ],
    )
    aw, am, ac = sc(w, wm, c, pid_p)

    out = pl.pallas_call(
        _fin_body,
        out_shape=jax.ShapeDtypeStruct((1, 1), jnp.float32),
    )(aw, am, ac)
    return out[0, 0]
